# baseline (speedup 1.0000x reference)
import math
import functools

import jax
import jax.numpy as jnp
from jax import lax
from jax.experimental import pallas as pl
from jax.experimental.pallas import tpu as pltpu
from jax.experimental.pallas import tpu_sc as plsc

IN_DIM = 258
OUT_DIM = 2048
SEQ_LEN = 1000

BM = 512
SL = 16
CH = 32


def _mm_body(x_ref, wt_ref, b_ref, id_ref, o_ref):
    xb = x_ref[...].astype(jnp.bfloat16)
    acc = jnp.dot(xb, wt_ref[...], preferred_element_type=jnp.float32)
    o_ref[...] = acc + b_ref[...] + id_ref[...].astype(jnp.float32)


def _matmul_add(x_flat, wt, b_aug, id_out):
    m = x_flat.shape[0]
    grid = (m // BM,)
    return pl.pallas_call(
        _mm_body,
        grid=grid,
        in_specs=[
            pl.BlockSpec((BM, IN_DIM), lambda i: (i, 0)),
            pl.BlockSpec((IN_DIM, OUT_DIM), lambda i: (0, 0)),
            pl.BlockSpec((1, OUT_DIM), lambda i: (0, 0)),
            pl.BlockSpec((BM, OUT_DIM), lambda i: (i, 0)),
        ],
        out_specs=pl.BlockSpec((BM, OUT_DIM), lambda i: (i, 0)),
        out_shape=jax.ShapeDtypeStruct((m, OUT_DIM), jnp.float32),
    )(x_flat, wt, b_aug, id_out)


@functools.partial(jax.jit, static_argnums=(2,))
def _sc_gather(emb_i, idx, m):
    info = plsc.get_sparse_core_info()
    nw = info.num_cores * info.num_subcores
    b_per_w = m // nw
    n_chunks = b_per_w // CH
    d_i = emb_i.shape[1]
    mesh = plsc.VectorSubcoreMesh(
        core_axis_name="c", subcore_axis_name="s", num_cores=info.num_cores
    )

    @functools.partial(
        pl.kernel,
        mesh=mesh,
        out_type=jax.ShapeDtypeStruct((m, d_i), jnp.int32),
        scratch_types=[
            pltpu.VMEM((CH,), jnp.int32),
            pltpu.VMEM((CH, d_i), jnp.int32),
            pltpu.SemaphoreType.DMA,
        ],
    )
    def k(emb_hbm, idx_hbm, out_hbm, idx_v, rows_v, sem):
        wid = lax.axis_index("s") * info.num_cores + lax.axis_index("c")
        base = wid * b_per_w

        def body(c, carry):
            off = base + c * CH
            pltpu.sync_copy(idx_hbm.at[pl.ds(off, CH)], idx_v)
            pltpu.async_copy(emb_hbm.at[idx_v], rows_v, sem).wait()
            pltpu.sync_copy(rows_v, out_hbm.at[pl.ds(off, CH)])
            return carry

        lax.fori_loop(0, n_chunks, body, 0)

    return k(emb_i, idx)


def kernel(x, W_epoch, emb, W_conf, b_conf):
    B, S, _ = x.shape
    m = B * S
    x_flat = x.reshape(m, IN_DIM)

    scale = math.sqrt(12.0) / float(SEQ_LEN)
    w_ep = W_epoch[:, 0]
    wt = jnp.concatenate(
        [
            (w_ep * scale)[None, :],
            jnp.zeros((1, OUT_DIM), jnp.float32),
            W_conf.T,
        ],
        axis=0,
    ).astype(jnp.bfloat16)
    b_aug = (b_conf - 0.5 * math.sqrt(12.0) * w_ep)[None, :]

    idx = x_flat[:, 1].astype(jnp.int32)
    emb_i = lax.bitcast_convert_type(
        emb.astype(jnp.bfloat16).reshape(emb.shape[0], OUT_DIM // 2, 2),
        jnp.int32,
    )
    id_i = _sc_gather(emb_i, idx, m)
    id_out = lax.bitcast_convert_type(id_i, jnp.bfloat16).reshape(m, OUT_DIM)

    out = _matmul_add(x_flat, wt, b_aug, id_out)
    return out.reshape(B, S, OUT_DIM)
